# baseline (speedup 1.0000x reference)
import jax
import jax.numpy as jnp
from jax.experimental import pallas as pl
from jax.experimental.pallas import tpu as pltpu

T = 16384
D = 4096
E = 64
TILE_T = 512
HALF = T // 2


def _router_kernel(xa_ref, xb_ref, w_ref,
                   wa_ref, ia_ref, wb_ref, ib_ref):
    w = w_ref[...]
    for x_ref, ow_ref, oi_ref in ((xa_ref, wa_ref, ia_ref),
                                  (xb_ref, wb_ref, ib_ref)):
        logits = jax.lax.dot_general(
            x_ref[...], w,
            dimension_numbers=(((1,), (1,)), ((), ())),
            preferred_element_type=jnp.float32,
        )
        m = jnp.max(logits, axis=-1)
        idx = jnp.argmax(logits, axis=-1).astype(jnp.int32)
        s = jnp.sum(jnp.exp(logits - m[:, None]), axis=-1)
        ow_ref[...] = 1.0 / s
        oi_ref[...] = idx


def kernel(x, W):
    grid = (HALF // TILE_T,)
    wa, ia, wb, ib = pl.pallas_call(
        _router_kernel,
        grid=grid,
        in_specs=[
            pl.BlockSpec((TILE_T, D), lambda i: (i, 0)),
            pl.BlockSpec((TILE_T, D), lambda i: (i + HALF // TILE_T, 0)),
            pl.BlockSpec((E, D), lambda i: (0, 0)),
        ],
        out_specs=[
            pl.BlockSpec((TILE_T,), lambda i: (i,)),
            pl.BlockSpec((TILE_T,), lambda i: (i,)),
            pl.BlockSpec((TILE_T,), lambda i: (i,)),
            pl.BlockSpec((TILE_T,), lambda i: (i,)),
        ],
        out_shape=[
            jax.ShapeDtypeStruct((HALF,), jnp.float32),
            jax.ShapeDtypeStruct((HALF,), jnp.int32),
            jax.ShapeDtypeStruct((HALF,), jnp.float32),
            jax.ShapeDtypeStruct((HALF,), jnp.int32),
        ],
        compiler_params=pltpu.CompilerParams(
            dimension_semantics=("parallel",),
        ),
    )(x, x, W)
    return (jnp.concatenate([wa, wb]), jnp.concatenate([ia, ib]))
